# baseline (speedup 1.0000x reference)
import functools

import jax
import jax.numpy as jnp
from jax import lax
from jax.experimental import pallas as pl
from jax.experimental.pallas import tpu as pltpu
from jax.experimental.pallas import tpu_sc as plsc

B, C, K, H, W = 8, 2, 1024, 512, 512
NUM_CELLS = 2 * B * C
LANES = 16
POINTS = 2 * K
CHUNK = 128
NCHUNK = POINTS // CHUNK
GROUPS = CHUNK // LANES

_mesh = plsc.VectorSubcoreMesh(core_axis_name="c", subcore_axis_name="s")

_DNUMS = lax.GatherDimensionNumbers(
    offset_dims=(), collapsed_slice_dims=(0,), start_index_map=(0,))


def _permute(v, idx):
    return lax.gather(v, idx[:, None], _DNUMS, slice_sizes=(1,),
                      mode=lax.GatherScatterMode.PROMISE_IN_BOUNDS)


@functools.partial(
    pl.kernel,
    out_type=jax.ShapeDtypeStruct((NUM_CELLS, LANES), jnp.float32),
    mesh=_mesh,
    scratch_types=[
        pltpu.VMEM((4 * K,), jnp.int32),
        pltpu.VMEM((NCHUNK, CHUNK), jnp.int32),
        pltpu.VMEM((NCHUNK, CHUNK), jnp.float32),
        pltpu.VMEM((LANES,), jnp.float32),
        pltpu.SemaphoreType.DMA,
    ],
)
def _bd_loss_sc(pred_hbm, ints_hbm, out_hbm, row_v, pidx_v, vals_v, acc_v, sem):
    cell = lax.axis_index("s") * 2 + lax.axis_index("c")
    comp = lax.div(cell, 16)
    bc = lax.rem(cell, 16)
    base = bc * (H * W)
    good_i = lax.max(1 - comp, lax.rem(cell, 2))

    pltpu.sync_copy(ints_hbm.at[cell], row_v)

    lane = lax.iota(jnp.int32, LANES)
    perm1 = (lane + 1) & 15
    perm_even = (lane & 7) * 2
    mask_a = 1 - (lane >> 3)
    mask_b = 1 - mask_a

    copies = []
    for j in range(NCHUNK):
        for h in range(GROUPS):
            o = j * 2 * CHUNK + h * 2 * LANES
            u0 = row_v[pl.ds(o, LANES)]
            u1 = row_v[pl.ds(o + LANES, LANES)]
            l0 = u0 * W + _permute(u0, perm1)
            l1 = u1 * W + _permute(u1, perm1)
            merged = (_permute(l0, perm_even) * mask_a
                      + _permute(l1, perm_even) * mask_b + base)
            pidx_v[j, pl.ds(h * LANES, LANES)] = merged
        cp = pltpu.make_async_copy(pred_hbm.at[pidx_v.at[j]], vals_v.at[j], sem)
        cp.start()
        copies.append(cp)
    for cp in copies:
        cp.wait()

    mask_even = (1 - (lane & 1)).astype(jnp.float32)
    flip = (jnp.maximum(1 - lane, 0) * good_i).astype(jnp.float32)
    acc = jnp.zeros((LANES,), jnp.float32)
    for j in range(NCHUNK):
        for h in range(GROUPS):
            w = vals_v[j, pl.ds(h * LANES, LANES)]
            d = w - _permute(w, perm1)
            d2 = d * d * mask_even
            if j == 0 and h == 0:
                d2 = d2 + flip * (1.0 - 2.0 * d2)
            acc = acc + d2

    acc_v[...] = acc
    pltpu.sync_copy(acc_v, out_hbm.at[cell])


def kernel(prediction, intervals_comp_0, intervals_comp_1):
    ints = jnp.concatenate([
        intervals_comp_0.reshape(B * C, 4 * K),
        intervals_comp_1.reshape(B * C, 4 * K),
    ])
    partials = _bd_loss_sc(prediction.reshape(-1), ints)
    return jnp.sum(partials)
